# Initial kernel scaffold; baseline (speedup 1.0000x reference)
import functools

import jax
import jax.numpy as jnp
from jax import lax
from jax.experimental import pallas as pl
from jax.experimental.pallas import tpu as pltpu
from jax.experimental.pallas import tpu_sc as plsc

N = 10000
E = 320000
D_IN = 128
D_HID = 64
D_OUT = 128

NC = 2
NS = 16
NW = NC * NS
CHUNK = 128
CPW = 79
EPW = CHUNK * CPW
E_PAD = NW * EPW
NPAD = 10016
RPT = NPAD // NS

ROW_BLK = 400


def _make_sc_agg(with_count):
  mesh = plsc.VectorSubcoreMesh(core_axis_name="c", subcore_axis_name="s")
  out_type = [jax.ShapeDtypeStruct((NC, NPAD, 4, 16), jnp.float32)]
  scratch = [
      pltpu.VMEM((CHUNK,), jnp.int32),
      pltpu.VMEM((CHUNK,), jnp.int32),
      pltpu.VMEM((CHUNK, 4, 16), jnp.float32),
      pltpu.VMEM((RPT, 4, 16), jnp.float32),
      pltpu.VMEM_SHARED((NPAD, 4, 16), jnp.float32),
      pltpu.SemaphoreType.DMA,
  ]
  if with_count:
    out_type.append(jax.ShapeDtypeStruct((NC, NPAD, 16), jnp.float32))
    scratch += [
        pltpu.VMEM((CHUNK, 16), jnp.float32),
        pltpu.VMEM((RPT, 16), jnp.float32),
        pltpu.VMEM_SHARED((NPAD, 16), jnp.float32),
    ]

  def body(p_hbm, src_hbm, dst_hbm, *rest):
    if with_count:
      (agg_out, cnt_out, idx_s, idx_d, rows, zrow, sh_agg, sem,
       ones, zcnt, sh_cnt) = rest
    else:
      agg_out, idx_s, idx_d, rows, zrow, sh_agg, sem = rest

    core = lax.axis_index("c")
    sub = lax.axis_index("s")
    w = sub * NC + core

    def zrow_body(i, carry):
      for j in range(4):
        zrow[i, j] = jnp.zeros((16,), jnp.float32)
      return carry
    lax.fori_loop(0, RPT, zrow_body, 0)
    pltpu.sync_copy(zrow, sh_agg.at[pl.ds(sub * RPT, RPT)])
    if with_count:
      def zcnt_body(i, carry):
        zcnt[i] = jnp.zeros((16,), jnp.float32)
        return carry
      lax.fori_loop(0, RPT, zcnt_body, 0)
      pltpu.sync_copy(zcnt, sh_cnt.at[pl.ds(sub * RPT, RPT)])

      def ones_body(i, carry):
        ones[i] = jnp.ones((16,), jnp.float32)
        return carry
      lax.fori_loop(0, CHUNK, ones_body, 0)
    plsc.subcore_barrier()

    def edge_body(c, carry):
      base = w * EPW + c * CHUNK
      pltpu.sync_copy(src_hbm.at[pl.ds(base, CHUNK)], idx_s)
      pltpu.sync_copy(dst_hbm.at[pl.ds(base, CHUNK)], idx_d)
      pltpu.async_copy(p_hbm.at[idx_s], rows, sem).wait()
      pltpu.sync_copy(rows, sh_agg.at[idx_d], add=True)
      if with_count:
        pltpu.sync_copy(ones, sh_cnt.at[idx_d], add=True)
      return carry
    lax.fori_loop(0, CPW, edge_body, 0)

    plsc.subcore_barrier()
    pltpu.sync_copy(sh_agg.at[pl.ds(sub * RPT, RPT)],
                    agg_out.at[core, pl.ds(sub * RPT, RPT)])
    if with_count:
      pltpu.sync_copy(sh_cnt.at[pl.ds(sub * RPT, RPT)],
                      cnt_out.at[core, pl.ds(sub * RPT, RPT)])

  return pl.kernel(body, out_type=tuple(out_type), mesh=mesh,
                   scratch_types=tuple(scratch))


_sc_agg_cnt = _make_sc_agg(with_count=True)
_sc_agg = _make_sc_agg(with_count=False)


def _mm_dual_body(x_ref, w_ref, p_ref, r_ref):
  xw = jnp.dot(x_ref[...], w_ref[...], preferred_element_type=jnp.float32)
  p_ref[...] = xw[:, :D_HID]
  r_ref[...] = xw[:, D_HID:]


def _mm_dual(x, w_cat):
  return pl.pallas_call(
      _mm_dual_body,
      grid=(N // ROW_BLK,),
      in_specs=[
          pl.BlockSpec((ROW_BLK, D_IN), lambda i: (i, 0)),
          pl.BlockSpec((D_IN, 2 * D_HID), lambda i: (0, 0)),
      ],
      out_specs=[
          pl.BlockSpec((ROW_BLK, D_HID), lambda i: (i, 0)),
          pl.BlockSpec((ROW_BLK, D_HID), lambda i: (i, 0)),
      ],
      out_shape=[
          jax.ShapeDtypeStruct((N, D_HID), jnp.float32),
          jax.ShapeDtypeStruct((N, D_HID), jnp.float32),
      ],
  )(x, w_cat)


def _combine1_body(aggp_ref, cntp_ref, r_ref, b_ref, h_ref):
  a = aggp_ref[0] + aggp_ref[1]
  c = jnp.maximum(cntp_ref[0, :, :1] + cntp_ref[1, :, :1], 1.0)
  h_ref[...] = jnp.maximum(a / c + b_ref[...] + r_ref[...], 0.0)


def _combine1(aggp, cntp, r1, b1):
  return pl.pallas_call(
      _combine1_body,
      grid=(N // ROW_BLK,),
      in_specs=[
          pl.BlockSpec((NC, ROW_BLK, D_HID), lambda i: (0, i, 0)),
          pl.BlockSpec((NC, ROW_BLK, 16), lambda i: (0, i, 0)),
          pl.BlockSpec((ROW_BLK, D_HID), lambda i: (i, 0)),
          pl.BlockSpec((1, D_HID), lambda i: (0, 0)),
      ],
      out_specs=pl.BlockSpec((ROW_BLK, D_HID), lambda i: (i, 0)),
      out_shape=jax.ShapeDtypeStruct((N, D_HID), jnp.float32),
  )(aggp, cntp, r1, b1)


def _combine2_body(aggp_ref, cntp_ref, h_ref, wl_ref, wr_ref, b_ref, out_ref):
  a = aggp_ref[0] + aggp_ref[1]
  c = jnp.maximum(cntp_ref[0, :, :1] + cntp_ref[1, :, :1], 1.0)
  out_ref[...] = (
      jnp.dot(a, wl_ref[...], preferred_element_type=jnp.float32) / c
      + b_ref[...]
      + jnp.dot(h_ref[...], wr_ref[...], preferred_element_type=jnp.float32))


def _combine2(aggp, cntp, h, w_l2, w_r2, b2):
  return pl.pallas_call(
      _combine2_body,
      grid=(N // ROW_BLK,),
      in_specs=[
          pl.BlockSpec((NC, ROW_BLK, D_HID), lambda i: (0, i, 0)),
          pl.BlockSpec((NC, ROW_BLK, 16), lambda i: (0, i, 0)),
          pl.BlockSpec((ROW_BLK, D_HID), lambda i: (i, 0)),
          pl.BlockSpec((D_HID, D_OUT), lambda i: (0, 0)),
          pl.BlockSpec((D_HID, D_OUT), lambda i: (0, 0)),
          pl.BlockSpec((1, D_OUT), lambda i: (0, 0)),
      ],
      out_specs=pl.BlockSpec((ROW_BLK, D_OUT), lambda i: (i, 0)),
      out_shape=jax.ShapeDtypeStruct((N, D_OUT), jnp.float32),
  )(aggp, cntp, h, w_l2, w_r2, b2)


def kernel(x, edge_index, W_l1, b_l1, W_r1, W_l2, b_l2, W_r2):
  src = edge_index[0].astype(jnp.int32)
  dst = edge_index[1].astype(jnp.int32)
  pad = E_PAD - E
  src_p = jnp.concatenate([src, jnp.zeros((pad,), jnp.int32)])
  dst_p = jnp.concatenate([dst, jnp.full((pad,), NPAD - 1, jnp.int32)])

  w_cat1 = jnp.concatenate([W_l1, W_r1], axis=1)
  p1, r1 = _mm_dual(x, w_cat1)

  aggp1, cntp = _sc_agg_cnt(p1.reshape(N, 4, 16), src_p, dst_p)
  h = _combine1(aggp1.reshape(NC, NPAD, D_HID), cntp, r1,
                b_l1.reshape(1, D_HID))

  aggp2 = _sc_agg(h.reshape(N, 4, 16), src_p, dst_p)
  out = _combine2(aggp2.reshape(NC, NPAD, D_HID), cntp, h,
                  W_l2, W_r2, b_l2.reshape(1, D_OUT))
  return out


# speedup vs baseline: 3.5848x; 3.5848x over previous
import functools

import jax
import jax.numpy as jnp
from jax import lax
from jax.experimental import pallas as pl
from jax.experimental.pallas import tpu as pltpu
from jax.experimental.pallas import tpu_sc as plsc

N = 10000
E = 320000
D_IN = 128
D_HID = 64
D_OUT = 128

NC = 2
NS = 16
NW = NC * NS
CHUNK = 128
CPW = 79
EPW = CHUNK * CPW
E_PAD = NW * EPW
NPAD = 10240
RPT = NPAD // NS

ROW_BLK = 400


def _make_sc_agg(with_count):
  mesh = plsc.VectorSubcoreMesh(core_axis_name="c", subcore_axis_name="s")
  out_type = [jax.ShapeDtypeStruct((NC, NPAD, 128), jnp.float32)]
  scratch = [
      pltpu.VMEM((CHUNK,), jnp.int32),
      pltpu.VMEM((CHUNK,), jnp.int32),
      pltpu.VMEM((CHUNK, 128), jnp.float32),
      pltpu.VMEM((16, 128), jnp.float32),
      pltpu.VMEM_SHARED((NPAD, 128), jnp.float32),
      pltpu.SemaphoreType.DMA,
  ]
  if with_count:
    out_type.append(jax.ShapeDtypeStruct((NW, NPAD), jnp.float32))
    scratch += [
        pltpu.VMEM((NPAD,), jnp.float32),
    ]

  def body(p_hbm, src_hbm, dst_hbm, *rest):
    if with_count:
      (agg_out, cnt_out, idx_s, idx_d, rows, zrow, sh_agg, sem,
       cnt_loc) = rest
    else:
      agg_out, idx_s, idx_d, rows, zrow, sh_agg, sem = rest

    core = lax.axis_index("c")
    sub = lax.axis_index("s")
    w = sub * NC + core

    zero16 = jnp.zeros((16,), jnp.float32)

    def zrow_body(i, carry):
      for j in range(8):
        zrow[i, 16 * j:16 * (j + 1)] = zero16
      return carry
    lax.fori_loop(0, 16, zrow_body, 0)

    def zcopy_body(k, carry):
      pltpu.sync_copy(zrow, sh_agg.at[pl.ds(sub * RPT + k * 16, 16)])
      return carry
    lax.fori_loop(0, RPT // 16, zcopy_body, 0)
    if with_count:
      def zcnt_body(i, carry):
        cnt_loc[pl.ds(i * 16, 16)] = zero16
        return carry
      lax.fori_loop(0, NPAD // 16, zcnt_body, 0)
    plsc.subcore_barrier()

    ones16 = jnp.ones((16,), jnp.float32)

    def edge_body(c, carry):
      base = w * EPW + c * CHUNK
      pltpu.sync_copy(src_hbm.at[pl.ds(base, CHUNK)], idx_s)
      pltpu.sync_copy(dst_hbm.at[pl.ds(base, CHUNK)], idx_d)
      pltpu.async_copy(p_hbm.at[idx_s], rows, sem).wait()
      pltpu.sync_copy(rows, sh_agg.at[idx_d], add=True)
      if with_count:
        for j in range(CHUNK // 16):
          dv = idx_d[pl.ds(16 * j, 16)]
          plsc.addupdate_scatter(cnt_loc, [dv], ones16)
      return carry
    lax.fori_loop(0, CPW, edge_body, 0)

    if with_count:
      pltpu.sync_copy(cnt_loc, cnt_out.at[w])
    plsc.subcore_barrier()
    pltpu.sync_copy(sh_agg.at[pl.ds(sub * RPT, RPT)],
                    agg_out.at[core, pl.ds(sub * RPT, RPT)])

  out_ty = tuple(out_type) if with_count else out_type[0]
  return pl.kernel(body, out_type=out_ty, mesh=mesh,
                   scratch_types=tuple(scratch),
                   compiler_params=pltpu.CompilerParams(
                       needs_layout_passes=False))


_sc_agg_cnt = _make_sc_agg(with_count=True)
_sc_agg = _make_sc_agg(with_count=False)


def _combine1_body(aggx_ref, cnt_ref, x_ref, wl1_ref, wr1_ref, b1_ref,
                   wl2_ref, h_ref, q_ref):
  a = aggx_ref[0] + aggx_ref[1]
  c = jnp.maximum(jnp.sum(cnt_ref[...], axis=0), 1.0)
  m = jnp.dot(a, wl1_ref[...], preferred_element_type=jnp.float32) / c
  h = jnp.maximum(
      m + b1_ref[...]
      + jnp.dot(x_ref[...], wr1_ref[...], preferred_element_type=jnp.float32),
      0.0)
  h_ref[...] = h
  q_ref[...] = jnp.dot(h, wl2_ref[...], preferred_element_type=jnp.float32)


def _combine1(aggx, cnt3, x, w_l1, w_r1, b1, w_l2):
  return pl.pallas_call(
      _combine1_body,
      grid=(N // ROW_BLK,),
      in_specs=[
          pl.BlockSpec((NC, ROW_BLK, D_IN), lambda i: (0, i, 0)),
          pl.BlockSpec((NW, ROW_BLK, 1), lambda i: (0, i, 0)),
          pl.BlockSpec((ROW_BLK, D_IN), lambda i: (i, 0)),
          pl.BlockSpec((D_IN, D_HID), lambda i: (0, 0)),
          pl.BlockSpec((D_IN, D_HID), lambda i: (0, 0)),
          pl.BlockSpec((1, D_HID), lambda i: (0, 0)),
          pl.BlockSpec((D_HID, D_OUT), lambda i: (0, 0)),
      ],
      out_specs=[
          pl.BlockSpec((ROW_BLK, D_HID), lambda i: (i, 0)),
          pl.BlockSpec((ROW_BLK, D_OUT), lambda i: (i, 0)),
      ],
      out_shape=[
          jax.ShapeDtypeStruct((N, D_HID), jnp.float32),
          jax.ShapeDtypeStruct((N, D_OUT), jnp.float32),
      ],
  )(aggx, cnt3, x, w_l1, w_r1, b1, w_l2)


def _combine2_body(aggq_ref, cnt_ref, h_ref, wr2_ref, b2_ref, out_ref):
  a = aggq_ref[0] + aggq_ref[1]
  c = jnp.maximum(jnp.sum(cnt_ref[...], axis=0), 1.0)
  out_ref[...] = (
      a / c + b2_ref[...]
      + jnp.dot(h_ref[...], wr2_ref[...], preferred_element_type=jnp.float32))


def _combine2(aggq, cnt3, h, w_r2, b2):
  return pl.pallas_call(
      _combine2_body,
      grid=(N // ROW_BLK,),
      in_specs=[
          pl.BlockSpec((NC, ROW_BLK, D_OUT), lambda i: (0, i, 0)),
          pl.BlockSpec((NW, ROW_BLK, 1), lambda i: (0, i, 0)),
          pl.BlockSpec((ROW_BLK, D_HID), lambda i: (i, 0)),
          pl.BlockSpec((D_HID, D_OUT), lambda i: (0, 0)),
          pl.BlockSpec((1, D_OUT), lambda i: (0, 0)),
      ],
      out_specs=pl.BlockSpec((ROW_BLK, D_OUT), lambda i: (i, 0)),
      out_shape=jax.ShapeDtypeStruct((N, D_OUT), jnp.float32),
  )(aggq, cnt3, h, w_r2, b2)


def kernel(x, edge_index, W_l1, b_l1, W_r1, W_l2, b_l2, W_r2):
  src = edge_index[0].astype(jnp.int32)
  dst = edge_index[1].astype(jnp.int32)
  pad = E_PAD - E
  src_p = jnp.concatenate([src, jnp.zeros((pad,), jnp.int32)])
  dst_p = jnp.concatenate([dst, jnp.full((pad,), NPAD - 1, jnp.int32)])

  aggx, cntp = _sc_agg_cnt(x, src_p, dst_p)
  cnt3 = cntp.reshape(NW, NPAD, 1)
  h, q = _combine1(aggx, cnt3, x, W_l1, W_r1, b_l1.reshape(1, D_HID), W_l2)

  aggq = _sc_agg(q, src_p, dst_p)
  out = _combine2(aggq, cnt3, h, W_r2, b_l2.reshape(1, D_OUT))
  return out
